# Initial kernel scaffold; baseline (speedup 1.0000x reference)
import jax
import jax.numpy as jnp
from jax.experimental import pallas as pl


def kernel(x, edge_index, w_steam, b_steam, w_blocks, w_lin, b_lin):
    raise NotImplementedError("write your pallas kernel here")


# speedup vs baseline: 1.3524x; 1.3524x over previous
import functools

import jax
import jax.numpy as jnp
import numpy as np
from jax.experimental import pallas as pl
from jax.experimental.pallas import tpu as pltpu

NUM_BLOCKS = 64
NCH = 32
ALPHA = 0.1
THETA = 0.5

_BLK = 2048


def _dense_body(agg_ref, h_ref, h0_ref, d2_ref, m_ref, out_ref):
    agg = agg_ref[...]
    h = h_ref[...]
    h0 = h0_ref[...]
    d2 = d2_ref[...]
    pre = (1.0 - ALPHA) * (agg + d2 * h) + ALPHA * h0
    out_ref[...] = jnp.dot(pre, m_ref[...], preferred_element_type=jnp.float32)


def _dense_update(agg, h, h0, d2, m):
    n = agg.shape[0]
    grid = (n + _BLK - 1) // _BLK
    return pl.pallas_call(
        _dense_body,
        grid=(grid,),
        in_specs=[
            pl.BlockSpec((_BLK, NCH), lambda i: (i, 0)),
            pl.BlockSpec((_BLK, NCH), lambda i: (i, 0)),
            pl.BlockSpec((_BLK, NCH), lambda i: (i, 0)),
            pl.BlockSpec((_BLK, 1), lambda i: (i, 0)),
            pl.BlockSpec((NCH, NCH), lambda i: (0, 0)),
        ],
        out_specs=pl.BlockSpec((_BLK, NCH), lambda i: (i, 0)),
        out_shape=jax.ShapeDtypeStruct((n, NCH), jnp.float32),
    )(agg, h, h0, d2, m)


def kernel(x, edge_index, w_steam, b_steam, w_blocks, w_lin, b_lin):
    n = x.shape[0]
    row = edge_index[0]
    col = edge_index[1]

    deg = jnp.zeros((n,), jnp.float32).at[col].add(1.0) + 1.0
    dinv = jax.lax.rsqrt(deg)
    norm = dinv[row] * dinv[col]
    d2 = (dinv * dinv)[:, None]

    h = x @ w_steam + b_steam
    h0 = h

    betas = jnp.log(THETA / jnp.arange(1, NUM_BLOCKS + 1, dtype=jnp.float32) + 1.0)
    eye = jnp.eye(NCH, dtype=jnp.float32)
    ms = (1.0 - betas)[:, None, None] * eye[None] + betas[:, None, None] * w_blocks

    def step(hc, m):
        msg = norm[:, None] * hc[row]
        agg = jnp.zeros_like(hc).at[col].add(msg)
        return _dense_update(agg, hc, h0, d2, m), None

    h, _ = jax.lax.scan(step, h, ms)
    return h @ w_lin + b_lin


# speedup vs baseline: 24.9711x; 18.4649x over previous
import functools

import jax
import jax.numpy as jnp
from jax import lax
from jax.experimental import pallas as pl
from jax.experimental.pallas import tpu as pltpu
from jax.experimental.pallas import tpu_sc as plsc

NUM_BLOCKS = 64
NCH = 32
ALPHA = 0.1
THETA = 0.5

N = 50000
E = 1600000
NT = 32
NPAD = 50048
NROWS_T = NPAD // 16
NACC = NPAD + 8
CH = 512
CHB = CH // 128
NCHUNKS = 3136
CPT = NCHUNKS // NT
EP = NCHUNKS * CH

_BLK = 2048


def _spmm_body(rowi_hbm, coli_hbm, g_hbm, z_hbm, out_hbm,
               rowv, colv, cidxv, stage, acc, sem):
    c = lax.axis_index("c")
    s = lax.axis_index("s")

    pltpu.sync_copy(z_hbm, acc.at[pl.ds(s * NROWS_T, NROWS_T)])
    plsc.subcore_barrier()

    cbase = (c * 16 + s) * CPT

    def chunk(k, carry):
        rb = pl.multiple_of((cbase + k) * CHB, CHB)
        pltpu.sync_copy(rowi_hbm.at[pl.ds(rb, CHB)], rowv)
        pltpu.sync_copy(coli_hbm.at[pl.ds(rb, CHB)], colv)
        cps = [pltpu.async_copy(g_hbm.at[rowv.at[j]],
                                stage.at[pl.ds(j * 128, 128)], sem)
               for j in range(CHB)]
        for j in range(CHB):
            for q in range(8):
                v = colv[j, pl.ds(q * 16, 16)]
                cidxv[j, pl.ds(q * 16, 16)] = jnp.minimum(v, jnp.int32(NPAD))
        for j in range(CHB):
            cps[j].wait()
            pltpu.sync_copy(stage.at[pl.ds(j * 128, 128)],
                            acc.at[cidxv.at[j]], add=True)
        return carry

    lax.fori_loop(0, CPT, chunk, 0)
    plsc.subcore_barrier()

    pltpu.sync_copy(acc.at[pl.ds(s * NROWS_T, NROWS_T)],
                    out_hbm.at[c, pl.ds(s * NROWS_T, NROWS_T)])


def _make_spmm():
    mesh = plsc.VectorSubcoreMesh(core_axis_name="c", subcore_axis_name="s")
    return pl.kernel(
        _spmm_body,
        out_type=jax.ShapeDtypeStruct((2, NPAD, NCH), jnp.float32),
        mesh=mesh,
        compiler_params=pltpu.CompilerParams(use_tc_tiling_on_sc=False),
        scratch_types=[
            pltpu.VMEM((CHB, 128), jnp.int32),
            pltpu.VMEM((CHB, 128), jnp.int32),
            pltpu.VMEM((CHB, 128), jnp.int32),
            pltpu.VMEM((CH, NCH), jnp.float32),
            pltpu.VMEM_SHARED((NACC, NCH), jnp.float32),
            pltpu.SemaphoreType.DMA,
        ],
    )


def _steam_body(x_ref, w_ref, b_ref, d1_ref, h0_ref, g0_ref):
    h = jnp.dot(x_ref[...], w_ref[...], preferred_element_type=jnp.float32)
    h = h + b_ref[...]
    h0_ref[...] = h
    g0_ref[...] = d1_ref[...] * h


def _steam(x, w_steam, b_steam, d1):
    n = x.shape[0]
    grid = (n + _BLK - 1) // _BLK
    return pl.pallas_call(
        _steam_body,
        grid=(grid,),
        in_specs=[
            pl.BlockSpec((_BLK, 128), lambda i: (i, 0)),
            pl.BlockSpec((128, NCH), lambda i: (0, 0)),
            pl.BlockSpec((1, NCH), lambda i: (0, 0)),
            pl.BlockSpec((_BLK, 1), lambda i: (i, 0)),
        ],
        out_specs=[
            pl.BlockSpec((_BLK, NCH), lambda i: (i, 0)),
            pl.BlockSpec((_BLK, NCH), lambda i: (i, 0)),
        ],
        out_shape=[
            jax.ShapeDtypeStruct((n, NCH), jnp.float32),
            jax.ShapeDtypeStruct((n, NCH), jnp.float32),
        ],
    )(x, w_steam, b_steam.reshape(1, NCH), d1)


def _dense_body(agg_ref, g_ref, h0_ref, d1_ref, m_ref, gout_ref):
    d1 = d1_ref[...]
    agg = agg_ref[0] + agg_ref[1]
    pre = (1.0 - ALPHA) * d1 * (agg + g_ref[...]) + ALPHA * h0_ref[...]
    h = jnp.dot(pre, m_ref[...], preferred_element_type=jnp.float32)
    gout_ref[...] = d1 * h


def _dense_update(agg2, g, h0, d1, m):
    n = g.shape[0]
    grid = (n + _BLK - 1) // _BLK
    return pl.pallas_call(
        _dense_body,
        grid=(grid,),
        in_specs=[
            pl.BlockSpec((2, _BLK, NCH), lambda i: (0, i, 0)),
            pl.BlockSpec((_BLK, NCH), lambda i: (i, 0)),
            pl.BlockSpec((_BLK, NCH), lambda i: (i, 0)),
            pl.BlockSpec((_BLK, 1), lambda i: (i, 0)),
            pl.BlockSpec((NCH, NCH), lambda i: (0, 0)),
        ],
        out_specs=pl.BlockSpec((_BLK, NCH), lambda i: (i, 0)),
        out_shape=jax.ShapeDtypeStruct((n, NCH), jnp.float32),
    )(agg2, g, h0, d1, m)


def _final_body(g_ref, s_ref, w_ref, b_ref, out_ref):
    h = g_ref[...] * s_ref[...]
    out_ref[...] = jnp.dot(h, w_ref[...], preferred_element_type=jnp.float32) + b_ref[...]


def _final(g, ssq, w_lin, b_lin):
    n = g.shape[0]
    grid = (n + _BLK - 1) // _BLK
    return pl.pallas_call(
        _final_body,
        grid=(grid,),
        in_specs=[
            pl.BlockSpec((_BLK, NCH), lambda i: (i, 0)),
            pl.BlockSpec((_BLK, 1), lambda i: (i, 0)),
            pl.BlockSpec((NCH, 1), lambda i: (0, 0)),
            pl.BlockSpec((1, 1), lambda i: (0, 0)),
        ],
        out_specs=pl.BlockSpec((_BLK, 1), lambda i: (i, 0)),
        out_shape=jax.ShapeDtypeStruct((n, 1), jnp.float32),
    )(g, ssq, w_lin, b_lin.reshape(1, 1))


def kernel(x, edge_index, w_steam, b_steam, w_blocks, w_lin, b_lin):
    row = edge_index[0]
    col = edge_index[1]

    rowp = jnp.concatenate([row, jnp.zeros((EP - E,), jnp.int32)])
    colp = jnp.concatenate([col, jnp.full((EP - E,), jnp.int32(2**30))])
    rowi2d = rowp.reshape(EP // 128, 128)
    coli2d = colp.reshape(EP // 128, 128)
    z = jnp.zeros((NROWS_T, NCH), jnp.float32)

    spmm = _make_spmm()

    aggd = spmm(rowi2d, coli2d, jnp.ones((N, NCH), jnp.float32), z)
    deg = aggd[0, :N, 0] + aggd[1, :N, 0] + 1.0
    dinv = lax.rsqrt(deg)
    d1 = dinv[:, None]
    ssq = jnp.sqrt(deg)[:, None]

    betas = jnp.log(THETA / jnp.arange(1, NUM_BLOCKS + 1, dtype=jnp.float32) + 1.0)
    eye = jnp.eye(NCH, dtype=jnp.float32)
    ms = (1.0 - betas)[:, None, None] * eye[None] + betas[:, None, None] * w_blocks

    h0, g0 = _steam(x, w_steam, b_steam, d1)

    def step(gc, m):
        aggf = spmm(rowi2d, coli2d, gc, z)
        agg2 = aggf[:, :N]
        return _dense_update(agg2, gc, h0, d1, m), None

    gfin, _ = lax.scan(step, g0, ms)
    return _final(gfin, ssq, w_lin, b_lin)


# speedup vs baseline: 29.4673x; 1.1801x over previous
import functools

import jax
import jax.numpy as jnp
from jax import lax
from jax.experimental import pallas as pl
from jax.experimental.pallas import tpu as pltpu
from jax.experimental.pallas import tpu_sc as plsc

NUM_BLOCKS = 64
NCH = 32
ALPHA = 0.1
THETA = 0.5

N = 50000
E = 1600000
NT = 32
NPAD = 50048
NROWS_T = NPAD // 16
NACC = NPAD + 8
CH = 256
CHB = CH // 128
SUP = 4
CPT = 196
NSUP = CPT // SUP
NCHUNKS = NT * CPT
EP = NCHUNKS * CH

_BLK = 2048


def _spmm_body(rowi_hbm, coli_hbm, g_hbm, z_hbm, out_hbm,
               rowv, colv, stage0, stage1, acc, gsem0, gsem1, ssem0, ssem1):
    c = lax.axis_index("c")
    s = lax.axis_index("s")

    pltpu.sync_copy(z_hbm, acc.at[pl.ds(s * NROWS_T, NROWS_T)])
    plsc.subcore_barrier()

    cbase = (c * 16 + s) * CPT
    stages = (stage0, stage1)
    gsems = (gsem0, gsem1)
    ssems = (ssem0, ssem1)

    def super_body(u, carry):
        sb = pl.multiple_of((cbase + u * SUP) * CHB, SUP * CHB)
        pltpu.sync_copy(rowi_hbm.at[pl.ds(sb, SUP * CHB)], rowv)
        pltpu.sync_copy(coli_hbm.at[pl.ds(sb, SUP * CHB)], colv)

        def fire_gather(k):
            b = k % 2
            return [pltpu.async_copy(g_hbm.at[rowv.at[k * CHB + j]],
                                     stages[b].at[pl.ds(j * 128, 128)],
                                     gsems[b])
                    for j in range(CHB)]

        def fire_scatter(k):
            b = k % 2
            return [pltpu.async_copy(stages[b].at[pl.ds(j * 128, 128)],
                                     acc.at[colv.at[k * CHB + j]],
                                     ssems[b], add=True)
                    for j in range(CHB)]

        g_pending = fire_gather(0)
        s_pending = [None, None]
        for k in range(SUP):
            b = k % 2
            if k < SUP - 1:
                nb = (k + 1) % 2
                if s_pending[nb] is not None:
                    for cp in s_pending[nb]:
                        cp.wait()
                    s_pending[nb] = None
                g_next = fire_gather(k + 1)
            for cp in g_pending:
                cp.wait()
            s_pending[b] = fire_scatter(k)
            if k < SUP - 1:
                g_pending = g_next
        for sp in s_pending:
            if sp is not None:
                for cp in sp:
                    cp.wait()
        return carry

    lax.fori_loop(0, NSUP, super_body, 0)
    plsc.subcore_barrier()

    pltpu.sync_copy(acc.at[pl.ds(s * NROWS_T, NROWS_T)],
                    out_hbm.at[c, pl.ds(s * NROWS_T, NROWS_T)])


def _make_spmm():
    mesh = plsc.VectorSubcoreMesh(core_axis_name="c", subcore_axis_name="s")
    return pl.kernel(
        _spmm_body,
        out_type=jax.ShapeDtypeStruct((2, NPAD, NCH), jnp.float32),
        mesh=mesh,
        compiler_params=pltpu.CompilerParams(use_tc_tiling_on_sc=False),
        scratch_types=[
            pltpu.VMEM((SUP * CHB, 128), jnp.int32),
            pltpu.VMEM((SUP * CHB, 128), jnp.int32),
            pltpu.VMEM((CH, NCH), jnp.float32),
            pltpu.VMEM((CH, NCH), jnp.float32),
            pltpu.VMEM_SHARED((NACC, NCH), jnp.float32),
            pltpu.SemaphoreType.DMA,
            pltpu.SemaphoreType.DMA,
            pltpu.SemaphoreType.DMA,
            pltpu.SemaphoreType.DMA,
        ],
    )


def _steam_body(x_ref, w_ref, b_ref, d1_ref, h0_ref, g0_ref):
    h = jnp.dot(x_ref[...], w_ref[...], preferred_element_type=jnp.float32)
    h = h + b_ref[...]
    h0_ref[...] = h
    g0_ref[...] = d1_ref[...] * h


def _steam(x, w_steam, b_steam, d1):
    n = x.shape[0]
    grid = (n + _BLK - 1) // _BLK
    return pl.pallas_call(
        _steam_body,
        grid=(grid,),
        in_specs=[
            pl.BlockSpec((_BLK, 128), lambda i: (i, 0)),
            pl.BlockSpec((128, NCH), lambda i: (0, 0)),
            pl.BlockSpec((1, NCH), lambda i: (0, 0)),
            pl.BlockSpec((_BLK, 1), lambda i: (i, 0)),
        ],
        out_specs=[
            pl.BlockSpec((_BLK, NCH), lambda i: (i, 0)),
            pl.BlockSpec((_BLK, NCH), lambda i: (i, 0)),
        ],
        out_shape=[
            jax.ShapeDtypeStruct((n, NCH), jnp.float32),
            jax.ShapeDtypeStruct((n, NCH), jnp.float32),
        ],
    )(x, w_steam, b_steam.reshape(1, NCH), d1)


def _dense_body(agg_ref, g_ref, h0_ref, d1_ref, m_ref, gout_ref):
    d1 = d1_ref[...]
    agg = agg_ref[0] + agg_ref[1]
    pre = (1.0 - ALPHA) * d1 * (agg + g_ref[...]) + ALPHA * h0_ref[...]
    h = jnp.dot(pre, m_ref[...], preferred_element_type=jnp.float32)
    gout_ref[...] = d1 * h


def _dense_update(agg2, g, h0, d1, m):
    n = g.shape[0]
    grid = (n + _BLK - 1) // _BLK
    return pl.pallas_call(
        _dense_body,
        grid=(grid,),
        in_specs=[
            pl.BlockSpec((2, _BLK, NCH), lambda i: (0, i, 0)),
            pl.BlockSpec((_BLK, NCH), lambda i: (i, 0)),
            pl.BlockSpec((_BLK, NCH), lambda i: (i, 0)),
            pl.BlockSpec((_BLK, 1), lambda i: (i, 0)),
            pl.BlockSpec((NCH, NCH), lambda i: (0, 0)),
        ],
        out_specs=pl.BlockSpec((_BLK, NCH), lambda i: (i, 0)),
        out_shape=jax.ShapeDtypeStruct((n, NCH), jnp.float32),
    )(agg2, g, h0, d1, m)


def _final_body(g_ref, s_ref, w_ref, b_ref, out_ref):
    h = g_ref[...] * s_ref[...]
    out_ref[...] = jnp.dot(h, w_ref[...], preferred_element_type=jnp.float32) + b_ref[...]


def _final(g, ssq, w_lin, b_lin):
    n = g.shape[0]
    grid = (n + _BLK - 1) // _BLK
    return pl.pallas_call(
        _final_body,
        grid=(grid,),
        in_specs=[
            pl.BlockSpec((_BLK, NCH), lambda i: (i, 0)),
            pl.BlockSpec((_BLK, 1), lambda i: (i, 0)),
            pl.BlockSpec((NCH, 1), lambda i: (0, 0)),
            pl.BlockSpec((1, 1), lambda i: (0, 0)),
        ],
        out_specs=pl.BlockSpec((_BLK, 1), lambda i: (i, 0)),
        out_shape=jax.ShapeDtypeStruct((n, 1), jnp.float32),
    )(g, ssq, w_lin, b_lin.reshape(1, 1))


def kernel(x, edge_index, w_steam, b_steam, w_blocks, w_lin, b_lin):
    row = edge_index[0]
    col = edge_index[1]

    npad = EP - E
    rowp = jnp.concatenate([row, jnp.zeros((npad,), jnp.int32)])
    colp = jnp.concatenate(
        [col, NPAD + (jnp.arange(npad, dtype=jnp.int32) % 8)])
    rowi2d = rowp.reshape(EP // 128, 128)
    coli2d = colp.reshape(EP // 128, 128)
    z = jnp.zeros((NROWS_T, NCH), jnp.float32)

    spmm = _make_spmm()

    aggd = spmm(rowi2d, coli2d, jnp.ones((N, NCH), jnp.float32), z)
    deg = aggd[0, :N, 0] + aggd[1, :N, 0] + 1.0
    dinv = lax.rsqrt(deg)
    d1 = dinv[:, None]
    ssq = jnp.sqrt(deg)[:, None]

    betas = jnp.log(THETA / jnp.arange(1, NUM_BLOCKS + 1, dtype=jnp.float32) + 1.0)
    eye = jnp.eye(NCH, dtype=jnp.float32)
    ms = (1.0 - betas)[:, None, None] * eye[None] + betas[:, None, None] * w_blocks

    h0, g0 = _steam(x, w_steam, b_steam, d1)

    def step(gc, m):
        aggf = spmm(rowi2d, coli2d, gc, z)
        return _dense_update(aggf, gc, h0, d1, m), None

    gfin, _ = lax.scan(step, g0, ms)
    return _final(gfin, ssq, w_lin, b_lin)


# speedup vs baseline: 35.1736x; 1.1936x over previous
import functools

import jax
import jax.numpy as jnp
from jax import lax
from jax.experimental import pallas as pl
from jax.experimental.pallas import tpu as pltpu
from jax.experimental.pallas import tpu_sc as plsc

NUM_BLOCKS = 64
NCH = 32
ALPHA = 0.1
THETA = 0.5

N = 50000
E = 1600000
NT = 32
NPAD = 50048
NROWS_T = NPAD // 16
NACC = NPAD + 64
CH = 256
CHB = CH // 128
SUP = 7
CPT = 196
NSUP = CPT // SUP
NCHUNKS = NT * CPT
EP = NCHUNKS * CH
SROWS = SUP * CHB

_BLK = 2048


def _spmm_body(rowi_hbm, coli_hbm, g_hbm, z_hbm, out_hbm,
               rowv, colv, stage0, stage1, acc,
               gsem0, gsem1, ssem0, ssem1, isem0, isem1):
    c = lax.axis_index("c")
    s = lax.axis_index("s")

    pltpu.sync_copy(z_hbm, acc.at[pl.ds(s * NROWS_T, NROWS_T)])
    plsc.subcore_barrier()

    cbase = (c * 16 + s) * CPT
    stages = (stage0, stage1)
    gsems = (gsem0, gsem1)
    ssems = (ssem0, ssem1)
    isems = (isem0, isem1)
    max_sb = (NCHUNKS - SUP) * CHB

    def idx_refs(u, b):
        sb = jnp.minimum((cbase + u * SUP) * CHB, max_sb)
        sb = pl.multiple_of(sb, 2)
        return [(rowi_hbm.at[pl.ds(sb, SROWS)],
                 rowv.at[pl.ds(b * SROWS, SROWS)]),
                (coli_hbm.at[pl.ds(sb, SROWS)],
                 colv.at[pl.ds(b * SROWS, SROWS)])]

    def fire_idx(u, b):
        for src, dst in idx_refs(u, b):
            pltpu.async_copy(src, dst, isems[b])

    def wait_idx(u, b):
        for src, dst in idx_refs(u, b):
            pltpu.make_async_copy(src, dst, isems[b]).wait()

    fire_idx(0, 0)

    def pair_body(p, carry):
        u0 = p * 2

        def fire_gather(k):
            ib = k // SUP
            return [pltpu.async_copy(
                g_hbm.at[rowv.at[ib * SROWS + (k - ib * SUP) * CHB + j]],
                stages[k % 2].at[pl.ds(j * 128, 128)], gsems[k % 2])
                for j in range(CHB)]

        def fire_scatter(k):
            ib = k // SUP
            return [pltpu.async_copy(
                stages[k % 2].at[pl.ds(j * 128, 128)],
                acc.at[colv.at[ib * SROWS + (k - ib * SUP) * CHB + j]],
                ssems[k % 2], add=True)
                for j in range(CHB)]

        wait_idx(u0, 0)
        fire_idx(u0 + 1, 1)
        g_pending = fire_gather(0)
        s_pending = [None, None]
        for k in range(2 * SUP):
            if k == SUP:
                wait_idx(u0 + 1, 1)
                fire_idx(u0 + 2, 0)
            if k < 2 * SUP - 1:
                nb = (k + 1) % 2
                if s_pending[nb] is not None:
                    for cp in s_pending[nb]:
                        cp.wait()
                    s_pending[nb] = None
                g_next = fire_gather(k + 1)
            for cp in g_pending:
                cp.wait()
            s_pending[k % 2] = fire_scatter(k)
            if k < 2 * SUP - 1:
                g_pending = g_next
        for sp in s_pending:
            if sp is not None:
                for cp in sp:
                    cp.wait()
        return carry

    lax.fori_loop(0, NSUP // 2, pair_body, 0)
    wait_idx(0, 0)
    plsc.subcore_barrier()

    pltpu.sync_copy(acc.at[pl.ds(s * NROWS_T, NROWS_T)],
                    out_hbm.at[c, pl.ds(s * NROWS_T, NROWS_T)])


def _make_spmm():
    mesh = plsc.VectorSubcoreMesh(core_axis_name="c", subcore_axis_name="s")
    return pl.kernel(
        _spmm_body,
        out_type=jax.ShapeDtypeStruct((2, NPAD, NCH), jnp.float32),
        mesh=mesh,
        compiler_params=pltpu.CompilerParams(use_tc_tiling_on_sc=False),
        scratch_types=[
            pltpu.VMEM((2 * SROWS, 128), jnp.int32),
            pltpu.VMEM((2 * SROWS, 128), jnp.int32),
            pltpu.VMEM((CH, NCH), jnp.float32),
            pltpu.VMEM((CH, NCH), jnp.float32),
            pltpu.VMEM_SHARED((NACC, NCH), jnp.float32),
            pltpu.SemaphoreType.DMA,
            pltpu.SemaphoreType.DMA,
            pltpu.SemaphoreType.DMA,
            pltpu.SemaphoreType.DMA,
            pltpu.SemaphoreType.DMA,
            pltpu.SemaphoreType.DMA,
        ],
    )


def _steam_body(x_ref, w_ref, b_ref, d1_ref, h0_ref, g0_ref):
    h = jnp.dot(x_ref[...], w_ref[...], preferred_element_type=jnp.float32)
    h = h + b_ref[...]
    h0_ref[...] = h
    g0_ref[...] = d1_ref[...] * h


def _steam(x, w_steam, b_steam, d1):
    n = x.shape[0]
    grid = (n + _BLK - 1) // _BLK
    return pl.pallas_call(
        _steam_body,
        grid=(grid,),
        in_specs=[
            pl.BlockSpec((_BLK, 128), lambda i: (i, 0)),
            pl.BlockSpec((128, NCH), lambda i: (0, 0)),
            pl.BlockSpec((1, NCH), lambda i: (0, 0)),
            pl.BlockSpec((_BLK, 1), lambda i: (i, 0)),
        ],
        out_specs=[
            pl.BlockSpec((_BLK, NCH), lambda i: (i, 0)),
            pl.BlockSpec((_BLK, NCH), lambda i: (i, 0)),
        ],
        out_shape=[
            jax.ShapeDtypeStruct((n, NCH), jnp.float32),
            jax.ShapeDtypeStruct((n, NCH), jnp.float32),
        ],
    )(x, w_steam, b_steam.reshape(1, NCH), d1)


def _dense_body(agg_ref, g_ref, h0_ref, d1_ref, m_ref, gout_ref):
    d1 = d1_ref[...]
    agg = agg_ref[0] + agg_ref[1]
    pre = (1.0 - ALPHA) * d1 * (agg + g_ref[...]) + ALPHA * h0_ref[...]
    h = jnp.dot(pre, m_ref[...], preferred_element_type=jnp.float32)
    gout_ref[...] = d1 * h


def _dense_update(agg2, g, h0, d1, m):
    n = g.shape[0]
    grid = (n + _BLK - 1) // _BLK
    return pl.pallas_call(
        _dense_body,
        grid=(grid,),
        in_specs=[
            pl.BlockSpec((2, _BLK, NCH), lambda i: (0, i, 0)),
            pl.BlockSpec((_BLK, NCH), lambda i: (i, 0)),
            pl.BlockSpec((_BLK, NCH), lambda i: (i, 0)),
            pl.BlockSpec((_BLK, 1), lambda i: (i, 0)),
            pl.BlockSpec((NCH, NCH), lambda i: (0, 0)),
        ],
        out_specs=pl.BlockSpec((_BLK, NCH), lambda i: (i, 0)),
        out_shape=jax.ShapeDtypeStruct((n, NCH), jnp.float32),
    )(agg2, g, h0, d1, m)


def _final_body(g_ref, s_ref, w_ref, b_ref, out_ref):
    h = g_ref[...] * s_ref[...]
    out_ref[...] = jnp.dot(h, w_ref[...], preferred_element_type=jnp.float32) + b_ref[...]


def _final(g, ssq, w_lin, b_lin):
    n = g.shape[0]
    grid = (n + _BLK - 1) // _BLK
    return pl.pallas_call(
        _final_body,
        grid=(grid,),
        in_specs=[
            pl.BlockSpec((_BLK, NCH), lambda i: (i, 0)),
            pl.BlockSpec((_BLK, 1), lambda i: (i, 0)),
            pl.BlockSpec((NCH, 1), lambda i: (0, 0)),
            pl.BlockSpec((1, 1), lambda i: (0, 0)),
        ],
        out_specs=pl.BlockSpec((_BLK, 1), lambda i: (i, 0)),
        out_shape=jax.ShapeDtypeStruct((n, 1), jnp.float32),
    )(g, ssq, w_lin, b_lin.reshape(1, 1))


def kernel(x, edge_index, w_steam, b_steam, w_blocks, w_lin, b_lin):
    row = edge_index[0]
    col = edge_index[1]

    npad = EP - E
    rowp = jnp.concatenate([row, jnp.zeros((npad,), jnp.int32)])
    colp = jnp.concatenate(
        [col, NPAD + (jnp.arange(npad, dtype=jnp.int32) % 64)])
    rowi2d = rowp.reshape(EP // 128, 128)
    coli2d = colp.reshape(EP // 128, 128)
    z = jnp.zeros((NROWS_T, NCH), jnp.float32)

    spmm = _make_spmm()

    aggd = spmm(rowi2d, coli2d, jnp.ones((N, NCH), jnp.float32), z)
    deg = aggd[0, :N, 0] + aggd[1, :N, 0] + 1.0
    dinv = lax.rsqrt(deg)
    d1 = dinv[:, None]
    ssq = jnp.sqrt(deg)[:, None]

    betas = jnp.log(THETA / jnp.arange(1, NUM_BLOCKS + 1, dtype=jnp.float32) + 1.0)
    eye = jnp.eye(NCH, dtype=jnp.float32)
    ms = (1.0 - betas)[:, None, None] * eye[None] + betas[:, None, None] * w_blocks

    h0, g0 = _steam(x, w_steam, b_steam, d1)

    def step(gc, m):
        aggf = spmm(rowi2d, coli2d, gc, z)
        return _dense_update(aggf, gc, h0, d1, m), None

    gfin, _ = lax.scan(step, g0, ms)
    return _final(gfin, ssq, w_lin, b_lin)


# speedup vs baseline: 37.0303x; 1.0528x over previous
import functools

import jax
import jax.numpy as jnp
from jax import lax
from jax.experimental import pallas as pl
from jax.experimental.pallas import tpu as pltpu
from jax.experimental.pallas import tpu_sc as plsc

NUM_BLOCKS = 64
NCH = 32
ALPHA = 0.1
THETA = 0.5

N = 50000
E = 1600000
NT = 32
NPAD = 50048
NROWS_T = NPAD // 16
NACC = NPAD + 64
CH = 256
CHB = CH // 128
SUP = 7
CPT = 196
NSUP = CPT // SUP
NCHUNKS = NT * CPT
EP = NCHUNKS * CH
SROWS = SUP * CHB

_BLK = 2048


def _spmm_body(rowi_hbm, coli_hbm, g_hbm, z_hbm, out_hbm,
               rowv, colv, stage0, stage1, acc,
               gsem0, gsem1, ssem0, ssem1, isem0, isem1):
    c = lax.axis_index("c")
    s = lax.axis_index("s")

    pltpu.sync_copy(z_hbm, acc.at[pl.ds(s * NROWS_T, NROWS_T)])
    plsc.subcore_barrier()

    cbase = (c * 16 + s) * CPT
    stages = (stage0, stage1)
    gsems = (gsem0, gsem1)
    ssems = (ssem0, ssem1)
    isems = (isem0, isem1)
    max_sb = (NCHUNKS - SUP) * CHB

    def idx_refs(u, b):
        sb = jnp.minimum((cbase + u * SUP) * CHB, max_sb)
        sb = pl.multiple_of(sb, 2)
        return [(rowi_hbm.at[pl.ds(sb, SROWS)],
                 rowv.at[pl.ds(b * SROWS, SROWS)]),
                (coli_hbm.at[pl.ds(sb, SROWS)],
                 colv.at[pl.ds(b * SROWS, SROWS)])]

    def fire_idx(u, b):
        for src, dst in idx_refs(u, b):
            pltpu.async_copy(src, dst, isems[b])

    def wait_idx(u, b):
        for src, dst in idx_refs(u, b):
            pltpu.make_async_copy(src, dst, isems[b]).wait()

    fire_idx(0, 0)

    def pair_body(p, carry):
        u0 = p * 2

        def fire_gather(k):
            ib = k // SUP
            return [pltpu.async_copy(
                g_hbm.at[rowv.at[ib * SROWS + (k - ib * SUP) * CHB + j]],
                stages[k % 2].at[pl.ds(j * 128, 128)], gsems[k % 2])
                for j in range(CHB)]

        def fire_scatter(k):
            ib = k // SUP
            return [pltpu.async_copy(
                stages[k % 2].at[pl.ds(j * 128, 128)],
                acc.at[colv.at[ib * SROWS + (k - ib * SUP) * CHB + j]],
                ssems[k % 2], add=True)
                for j in range(CHB)]

        wait_idx(u0, 0)
        fire_idx(u0 + 1, 1)
        g_pending = fire_gather(0)
        s_pending = [None, None]
        for k in range(2 * SUP):
            if k == SUP:
                wait_idx(u0 + 1, 1)
                fire_idx(u0 + 2, 0)
            if k < 2 * SUP - 1:
                nb = (k + 1) % 2
                if s_pending[nb] is not None:
                    for cp in s_pending[nb]:
                        cp.wait()
                    s_pending[nb] = None
                g_next = fire_gather(k + 1)
            for cp in g_pending:
                cp.wait()
            s_pending[k % 2] = fire_scatter(k)
            if k < 2 * SUP - 1:
                g_pending = g_next
        for sp in s_pending:
            if sp is not None:
                for cp in sp:
                    cp.wait()
        return carry

    lax.fori_loop(0, NSUP // 2, pair_body, 0)
    wait_idx(0, 0)
    plsc.subcore_barrier()

    pltpu.sync_copy(acc.at[pl.ds(s * NROWS_T, NROWS_T)],
                    out_hbm.at[c, pl.ds(s * NROWS_T, NROWS_T)])


def _make_spmm():
    mesh = plsc.VectorSubcoreMesh(core_axis_name="c", subcore_axis_name="s")
    return pl.kernel(
        _spmm_body,
        out_type=jax.ShapeDtypeStruct((2, NPAD, NCH), jnp.float32),
        mesh=mesh,
        compiler_params=pltpu.CompilerParams(use_tc_tiling_on_sc=False),
        scratch_types=[
            pltpu.VMEM((2 * SROWS, 128), jnp.int32),
            pltpu.VMEM((2 * SROWS, 128), jnp.int32),
            pltpu.VMEM((CH, NCH), jnp.float32),
            pltpu.VMEM((CH, NCH), jnp.float32),
            pltpu.VMEM_SHARED((NACC, NCH), jnp.float32),
            pltpu.SemaphoreType.DMA,
            pltpu.SemaphoreType.DMA,
            pltpu.SemaphoreType.DMA,
            pltpu.SemaphoreType.DMA,
            pltpu.SemaphoreType.DMA,
            pltpu.SemaphoreType.DMA,
        ],
    )


def _steam_body(x_ref, w_ref, b_ref, d1_ref, h0_ref, g0_ref):
    h = jnp.dot(x_ref[...], w_ref[...], preferred_element_type=jnp.float32)
    h = h + b_ref[...]
    h0_ref[...] = h
    g0_ref[...] = d1_ref[...] * h


def _steam(x, w_steam, b_steam, d1):
    n = x.shape[0]
    grid = (n + _BLK - 1) // _BLK
    return pl.pallas_call(
        _steam_body,
        grid=(grid,),
        in_specs=[
            pl.BlockSpec((_BLK, 128), lambda i: (i, 0)),
            pl.BlockSpec((128, NCH), lambda i: (0, 0)),
            pl.BlockSpec((1, NCH), lambda i: (0, 0)),
            pl.BlockSpec((_BLK, 1), lambda i: (i, 0)),
        ],
        out_specs=[
            pl.BlockSpec((_BLK, NCH), lambda i: (i, 0)),
            pl.BlockSpec((_BLK, NCH), lambda i: (i, 0)),
        ],
        out_shape=[
            jax.ShapeDtypeStruct((n, NCH), jnp.float32),
            jax.ShapeDtypeStruct((n, NCH), jnp.float32),
        ],
    )(x, w_steam, b_steam.reshape(1, NCH), d1)


def _dense_body(agg_ref, g_ref, h0_ref, d1_ref, m_ref, gout_ref):
    d1 = d1_ref[...]
    agg = agg_ref[0] + agg_ref[1]
    pre = (1.0 - ALPHA) * d1 * (agg + g_ref[...]) + ALPHA * h0_ref[...]
    h = jnp.dot(pre, m_ref[...], preferred_element_type=jnp.float32)
    gout_ref[...] = d1 * h


def _dense_update(agg2, g, h0, d1, m):
    n = g.shape[0]
    grid = (n + _BLK - 1) // _BLK
    return pl.pallas_call(
        _dense_body,
        grid=(grid,),
        in_specs=[
            pl.BlockSpec((2, _BLK, NCH), lambda i: (0, i, 0)),
            pl.BlockSpec((_BLK, NCH), lambda i: (i, 0)),
            pl.BlockSpec((_BLK, NCH), lambda i: (i, 0)),
            pl.BlockSpec((_BLK, 1), lambda i: (i, 0)),
            pl.BlockSpec((NCH, NCH), lambda i: (0, 0)),
        ],
        out_specs=pl.BlockSpec((_BLK, NCH), lambda i: (i, 0)),
        out_shape=jax.ShapeDtypeStruct((n, NCH), jnp.float32),
    )(agg2, g, h0, d1, m)


def _final_body(g_ref, s_ref, w_ref, b_ref, out_ref):
    h = g_ref[...] * s_ref[...]
    out_ref[...] = jnp.dot(h, w_ref[...], preferred_element_type=jnp.float32) + b_ref[...]


def _final(g, ssq, w_lin, b_lin):
    n = g.shape[0]
    grid = (n + _BLK - 1) // _BLK
    return pl.pallas_call(
        _final_body,
        grid=(grid,),
        in_specs=[
            pl.BlockSpec((_BLK, NCH), lambda i: (i, 0)),
            pl.BlockSpec((_BLK, 1), lambda i: (i, 0)),
            pl.BlockSpec((NCH, 1), lambda i: (0, 0)),
            pl.BlockSpec((1, 1), lambda i: (0, 0)),
        ],
        out_specs=pl.BlockSpec((_BLK, 1), lambda i: (i, 0)),
        out_shape=jax.ShapeDtypeStruct((n, 1), jnp.float32),
    )(g, ssq, w_lin, b_lin.reshape(1, 1))


def kernel(x, edge_index, w_steam, b_steam, w_blocks, w_lin, b_lin):
    row = edge_index[0]
    col = edge_index[1]

    npad = EP - E
    rowp = jnp.concatenate([row, jnp.zeros((npad,), jnp.int32)])
    colp = jnp.concatenate(
        [col, NPAD + (jnp.arange(npad, dtype=jnp.int32) % 64)])
    rowi2d = rowp.reshape(EP // 128, 128)
    coli2d = colp.reshape(EP // 128, 128)
    z = jnp.zeros((NROWS_T, NCH), jnp.float32)

    spmm = _make_spmm()

    aggd = spmm(rowi2d, coli2d, jnp.ones((N, NCH), jnp.float32), z)
    deg = aggd[0, :N, 0] + aggd[1, :N, 0] + 1.0
    dinv = lax.rsqrt(deg)
    d1 = dinv[:, None]
    ssq = jnp.sqrt(deg)[:, None]

    betas = jnp.log(THETA / jnp.arange(1, NUM_BLOCKS + 1, dtype=jnp.float32) + 1.0)
    eye = jnp.eye(NCH, dtype=jnp.float32)
    ms = (1.0 - betas)[:, None, None] * eye[None] + betas[:, None, None] * w_blocks

    h0, g0 = _steam(x, w_steam, b_steam, d1)

    g = g0
    for l in range(NUM_BLOCKS):
        aggf = spmm(rowi2d, coli2d, g, z)
        g = _dense_update(aggf, g, h0, d1, ms[l])
    return _final(g, ssq, w_lin, b_lin)


# speedup vs baseline: 53.5992x; 1.4474x over previous
import functools

import jax
import jax.numpy as jnp
from jax import lax
from jax.experimental import pallas as pl
from jax.experimental.pallas import tpu as pltpu
from jax.experimental.pallas import tpu_sc as plsc

NUM_BLOCKS = 64
NCH = 32
ALPHA = 0.1
THETA = 0.5

N = 50000
E = 1600000
NT = 32
NPAD = 50048
NP4 = NPAD // 4
NROWS_T = NPAD // 16
NACC = NPAD + 64
CH = 256
CHB = CH // 128
SUP = 7
CPT0 = 224
CPT1 = 168
NCHUNKS = 16 * (CPT0 + CPT1)
EP = NCHUNKS * CH
SROWS = SUP * CHB

_BLKP = 1024


def _spmm_body(rowi_hbm, coli_hbm, g_hbm, z_hbm, out_hbm,
               rowv, colv, stage0, stage1, acc,
               gsem0, gsem1, ssem0, ssem1, isem0, isem1):
    c = lax.axis_index("c")
    s = lax.axis_index("s")

    pltpu.sync_copy(z_hbm, acc.at[pl.ds(s * NROWS_T, NROWS_T)])
    plsc.subcore_barrier()

    cbase = jnp.where(c == 0, s * CPT0, 16 * CPT0 + s * CPT1)
    npairs = jnp.where(c == 0, CPT0 // (2 * SUP), CPT1 // (2 * SUP))
    stages = (stage0, stage1)
    gsems = (gsem0, gsem1)
    ssems = (ssem0, ssem1)
    isems = (isem0, isem1)
    max_sb = (NCHUNKS - SUP) * CHB

    def idx_refs(u, b):
        sb = jnp.minimum((cbase + u * SUP) * CHB, max_sb)
        sb = pl.multiple_of(sb, 2)
        return [(rowi_hbm.at[pl.ds(sb, SROWS)],
                 rowv.at[pl.ds(b * SROWS, SROWS)]),
                (coli_hbm.at[pl.ds(sb, SROWS)],
                 colv.at[pl.ds(b * SROWS, SROWS)])]

    def fire_idx(u, b):
        for src, dst in idx_refs(u, b):
            pltpu.async_copy(src, dst, isems[b])

    def wait_idx(u, b):
        for src, dst in idx_refs(u, b):
            pltpu.make_async_copy(src, dst, isems[b]).wait()

    fire_idx(0, 0)

    def pair_body(p, carry):
        u0 = p * 2

        def fire_gather(k):
            ib = k // SUP
            return [pltpu.async_copy(
                g_hbm.at[rowv.at[ib * SROWS + (k - ib * SUP) * CHB + j]],
                stages[k % 2].at[pl.ds(j * 128, 128)], gsems[k % 2])
                for j in range(CHB)]

        def fire_scatter(k):
            ib = k // SUP
            return [pltpu.async_copy(
                stages[k % 2].at[pl.ds(j * 128, 128)],
                acc.at[colv.at[ib * SROWS + (k - ib * SUP) * CHB + j]],
                ssems[k % 2], add=True)
                for j in range(CHB)]

        wait_idx(u0, 0)
        fire_idx(u0 + 1, 1)
        g_pending = fire_gather(0)
        s_pending = [None, None]
        for k in range(2 * SUP):
            if k == SUP:
                wait_idx(u0 + 1, 1)
                fire_idx(u0 + 2, 0)
            if k < 2 * SUP - 1:
                nb = (k + 1) % 2
                if s_pending[nb] is not None:
                    for cp in s_pending[nb]:
                        cp.wait()
                    s_pending[nb] = None
                g_next = fire_gather(k + 1)
            for cp in g_pending:
                cp.wait()
            s_pending[k % 2] = fire_scatter(k)
            if k < 2 * SUP - 1:
                g_pending = g_next
        for sp in s_pending:
            if sp is not None:
                for cp in sp:
                    cp.wait()
        return carry

    lax.fori_loop(0, npairs, pair_body, 0)
    wait_idx(0, 0)
    plsc.subcore_barrier()

    pltpu.sync_copy(acc.at[pl.ds(s * NROWS_T, NROWS_T)],
                    out_hbm.at[c, pl.ds(s * NROWS_T, NROWS_T)])


def _make_spmm():
    mesh = plsc.VectorSubcoreMesh(core_axis_name="c", subcore_axis_name="s")
    return pl.kernel(
        _spmm_body,
        out_type=jax.ShapeDtypeStruct((2, NPAD, NCH), jnp.float32),
        mesh=mesh,
        compiler_params=pltpu.CompilerParams(use_tc_tiling_on_sc=False),
        scratch_types=[
            pltpu.VMEM((2 * SROWS, 128), jnp.int32),
            pltpu.VMEM((2 * SROWS, 128), jnp.int32),
            pltpu.VMEM((CH, NCH), jnp.float32),
            pltpu.VMEM((CH, NCH), jnp.float32),
            pltpu.VMEM_SHARED((NACC, NCH), jnp.float32),
            pltpu.SemaphoreType.DMA,
            pltpu.SemaphoreType.DMA,
            pltpu.SemaphoreType.DMA,
            pltpu.SemaphoreType.DMA,
            pltpu.SemaphoreType.DMA,
            pltpu.SemaphoreType.DMA,
        ],
    )


def _steam_body(x_ref, w_ref, b_ref, h0_ref):
    h = jnp.dot(x_ref[...], w_ref[...], preferred_element_type=jnp.float32)
    h0_ref[...] = h + b_ref[...]


def _steam(x, w_steam, b_steam):
    n = x.shape[0]
    blk = 2048
    grid = (n + blk - 1) // blk
    return pl.pallas_call(
        _steam_body,
        grid=(grid,),
        in_specs=[
            pl.BlockSpec((blk, 128), lambda i: (i, 0)),
            pl.BlockSpec((128, NCH), lambda i: (0, 0)),
            pl.BlockSpec((1, NCH), lambda i: (0, 0)),
        ],
        out_specs=pl.BlockSpec((blk, NCH), lambda i: (i, 0)),
        out_shape=jax.ShapeDtypeStruct((n, NCH), jnp.float32),
    )(x, w_steam, b_steam.reshape(1, NCH))


def _dense_body(agg_ref, g_ref, h0_ref, d1_ref, m_ref, gout_ref):
    d1 = d1_ref[...]
    agg = agg_ref[0] + agg_ref[1]
    pre = (1.0 - ALPHA) * d1 * (agg + g_ref[...]) + ALPHA * h0_ref[...]
    h = jnp.dot(pre, m_ref[...], preferred_element_type=jnp.float32)
    gout_ref[...] = d1 * h


def _dense_update(agg2p, gp, h0p, d1p, m4):
    grid = (NP4 + _BLKP - 1) // _BLKP
    return pl.pallas_call(
        _dense_body,
        grid=(grid,),
        in_specs=[
            pl.BlockSpec((2, _BLKP, 128), lambda i: (0, i, 0)),
            pl.BlockSpec((_BLKP, 128), lambda i: (i, 0)),
            pl.BlockSpec((_BLKP, 128), lambda i: (i, 0)),
            pl.BlockSpec((_BLKP, 128), lambda i: (i, 0)),
            pl.BlockSpec((128, 128), lambda i: (0, 0)),
        ],
        out_specs=pl.BlockSpec((_BLKP, 128), lambda i: (i, 0)),
        out_shape=jax.ShapeDtypeStruct((NP4, 128), jnp.float32),
    )(agg2p, gp, h0p, d1p, m4)


def _final_body(g_ref, s_ref, w_ref, b_ref, out_ref):
    h = g_ref[...] * s_ref[...]
    out_ref[...] = jnp.dot(h, w_ref[...], preferred_element_type=jnp.float32) + b_ref[...]


def _final(gp, ssqp, w4, b_lin):
    grid = (NP4 + _BLKP - 1) // _BLKP
    return pl.pallas_call(
        _final_body,
        grid=(grid,),
        in_specs=[
            pl.BlockSpec((_BLKP, 128), lambda i: (i, 0)),
            pl.BlockSpec((_BLKP, 128), lambda i: (i, 0)),
            pl.BlockSpec((128, 4), lambda i: (0, 0)),
            pl.BlockSpec((1, 1), lambda i: (0, 0)),
        ],
        out_specs=pl.BlockSpec((_BLKP, 4), lambda i: (i, 0)),
        out_shape=jax.ShapeDtypeStruct((NP4, 4), jnp.float32),
    )(gp, ssqp, w4, b_lin.reshape(1, 1))


def kernel(x, edge_index, w_steam, b_steam, w_blocks, w_lin, b_lin):
    row = edge_index[0]
    col = edge_index[1]

    npade = EP - E
    rowp = jnp.concatenate([row, jnp.zeros((npade,), jnp.int32)])
    colp = jnp.concatenate(
        [col, NPAD + (jnp.arange(npade, dtype=jnp.int32) % 64)])
    rowi2d = rowp.reshape(EP // 128, 128)
    coli2d = colp.reshape(EP // 128, 128)
    z = jnp.zeros((NROWS_T, NCH), jnp.float32)

    spmm = _make_spmm()

    aggd = spmm(rowi2d, coli2d, jnp.ones((NPAD, NCH), jnp.float32), z)
    deg = aggd[0, :N, 0] + aggd[1, :N, 0] + 1.0
    dinv = jnp.pad(lax.rsqrt(deg), (0, NPAD - N))
    d1p = jnp.repeat(dinv, NCH).reshape(NP4, 128)
    ssqp = jnp.repeat(jnp.pad(jnp.sqrt(deg), (0, NPAD - N)), NCH).reshape(NP4, 128)

    betas = jnp.log(THETA / jnp.arange(1, NUM_BLOCKS + 1, dtype=jnp.float32) + 1.0)
    eye = jnp.eye(NCH, dtype=jnp.float32)
    ms = (1.0 - betas)[:, None, None] * eye[None] + betas[:, None, None] * w_blocks
    eye4 = jnp.eye(4, dtype=jnp.float32)
    m4s = jax.vmap(lambda m: jnp.kron(eye4, m))(ms)
    w4 = jnp.kron(eye4, w_lin)

    h0 = _steam(x, w_steam, b_steam)
    h0p = jnp.pad(h0, ((0, NPAD - N), (0, 0))).reshape(NP4, 128)
    gp = d1p * h0p

    for l in range(NUM_BLOCKS):
        aggf = spmm(rowi2d, coli2d, gp.reshape(NPAD, NCH), z)
        aggp = aggf.reshape(2, NP4, 128)
        gp = _dense_update(aggp, gp, h0p, d1p, m4s[l])

    out4 = _final(gp, ssqp, w4, b_lin)
    return out4.reshape(NPAD, 1)[:N]


# speedup vs baseline: 55.1933x; 1.0297x over previous
import functools

import jax
import jax.numpy as jnp
from jax import lax
from jax.experimental import pallas as pl
from jax.experimental.pallas import tpu as pltpu
from jax.experimental.pallas import tpu_sc as plsc

NUM_BLOCKS = 64
NCH = 32
ALPHA = 0.1
THETA = 0.5

N = 50000
E = 1600000
NT = 32
NPAD = 50048
NP4 = NPAD // 4
NROWS_T = NPAD // 16
NACC = NPAD + 64
CH = 256
CHB = CH // 128
SUP = 7
CPT0 = 224
CPT1 = 168
NCHUNKS = 16 * (CPT0 + CPT1)
EP = NCHUNKS * CH
SROWS = SUP * CHB

_BLKP = 1024


def _spmm_body(rowi_hbm, coli_hbm, g_hbm, z_hbm, out_hbm,
               rowv, colv, stage0, stage1, acc,
               gsem0, gsem1, ssem0, ssem1, isem0, isem1):
    c = lax.axis_index("c")
    s = lax.axis_index("s")

    cbase = jnp.where(c == 0, s * CPT0, 16 * CPT0 + s * CPT1)
    npairs = jnp.where(c == 0, CPT0 // (2 * SUP), CPT1 // (2 * SUP))
    stages = (stage0, stage1)
    gsems = (gsem0, gsem1)
    ssems = (ssem0, ssem1)
    isems = (isem0, isem1)
    max_sb = (NCHUNKS - SUP) * CHB

    def idx_refs(u, b):
        sb = jnp.minimum((cbase + u * SUP) * CHB, max_sb)
        sb = pl.multiple_of(sb, 2)
        return [(rowi_hbm.at[pl.ds(sb, SROWS)],
                 rowv.at[pl.ds(b * SROWS, SROWS)]),
                (coli_hbm.at[pl.ds(sb, SROWS)],
                 colv.at[pl.ds(b * SROWS, SROWS)])]

    def fire_idx(u, b):
        for src, dst in idx_refs(u, b):
            pltpu.async_copy(src, dst, isems[b])

    def wait_idx(u, b):
        for src, dst in idx_refs(u, b):
            pltpu.make_async_copy(src, dst, isems[b]).wait()

    def scatter_refs(k):
        return [(stages[k % 2].at[pl.ds(j * 128, 128)],
                 acc.at[colv.at[j]]) for j in range(CHB)]

    def wait_scatter(k):
        for src, dst in scatter_refs(k):
            pltpu.make_async_copy(src, dst, ssems[k % 2]).wait()

    fire_idx(0, 0)

    pltpu.sync_copy(z_hbm, acc.at[pl.ds(s * NROWS_T, NROWS_T)])
    plsc.subcore_barrier()

    def pair_body(p, carry):
        u0 = p * 2

        def fire_gather(k):
            ib = k // SUP
            return [pltpu.async_copy(
                g_hbm.at[rowv.at[ib * SROWS + (k - ib * SUP) * CHB + j]],
                stages[k % 2].at[pl.ds(j * 128, 128)], gsems[k % 2])
                for j in range(CHB)]

        def fire_scatter(k):
            ib = k // SUP
            return [pltpu.async_copy(
                stages[k % 2].at[pl.ds(j * 128, 128)],
                acc.at[colv.at[ib * SROWS + (k - ib * SUP) * CHB + j]],
                ssems[k % 2], add=True)
                for j in range(CHB)]

        wait_idx(u0, 0)
        fire_idx(u0 + 1, 1)
        @pl.when(p > 0)
        def _():
            wait_scatter(0)
        g_pending = fire_gather(0)
        s_pending = [None, None]
        for k in range(2 * SUP):
            if k == SUP:
                wait_idx(u0 + 1, 1)
                fire_idx(u0 + 2, 0)
            if k < 2 * SUP - 1:
                nb = (k + 1) % 2
                if k == 0:
                    @pl.when(p > 0)
                    def _():
                        wait_scatter(1)
                elif s_pending[nb] is not None:
                    for cp in s_pending[nb]:
                        cp.wait()
                    s_pending[nb] = None
                g_next = fire_gather(k + 1)
            for cp in g_pending:
                cp.wait()
            s_pending[k % 2] = fire_scatter(k)
            if k < 2 * SUP - 1:
                g_pending = g_next
        return carry

    lax.fori_loop(0, npairs, pair_body, 0)
    wait_scatter(0)
    wait_scatter(1)
    wait_idx(0, 0)
    plsc.subcore_barrier()

    pltpu.sync_copy(acc.at[pl.ds(s * NROWS_T, NROWS_T)],
                    out_hbm.at[c, pl.ds(s * NROWS_T, NROWS_T)])


def _make_spmm():
    mesh = plsc.VectorSubcoreMesh(core_axis_name="c", subcore_axis_name="s")
    return pl.kernel(
        _spmm_body,
        out_type=jax.ShapeDtypeStruct((2, NPAD, NCH), jnp.float32),
        mesh=mesh,
        compiler_params=pltpu.CompilerParams(use_tc_tiling_on_sc=False),
        scratch_types=[
            pltpu.VMEM((2 * SROWS, 128), jnp.int32),
            pltpu.VMEM((2 * SROWS, 128), jnp.int32),
            pltpu.VMEM((CH, NCH), jnp.float32),
            pltpu.VMEM((CH, NCH), jnp.float32),
            pltpu.VMEM_SHARED((NACC, NCH), jnp.float32),
            pltpu.SemaphoreType.DMA,
            pltpu.SemaphoreType.DMA,
            pltpu.SemaphoreType.DMA,
            pltpu.SemaphoreType.DMA,
            pltpu.SemaphoreType.DMA,
            pltpu.SemaphoreType.DMA,
        ],
    )


def _steam_body(x_ref, w_ref, b_ref, h0_ref):
    h = jnp.dot(x_ref[...], w_ref[...], preferred_element_type=jnp.float32)
    h0_ref[...] = h + b_ref[...]


def _steam(x, w_steam, b_steam):
    n = x.shape[0]
    blk = 2048
    grid = (n + blk - 1) // blk
    return pl.pallas_call(
        _steam_body,
        grid=(grid,),
        in_specs=[
            pl.BlockSpec((blk, 128), lambda i: (i, 0)),
            pl.BlockSpec((128, NCH), lambda i: (0, 0)),
            pl.BlockSpec((1, NCH), lambda i: (0, 0)),
        ],
        out_specs=pl.BlockSpec((blk, NCH), lambda i: (i, 0)),
        out_shape=jax.ShapeDtypeStruct((n, NCH), jnp.float32),
    )(x, w_steam, b_steam.reshape(1, NCH))


def _dense_body(agg_ref, g_ref, h0_ref, d1_ref, m_ref, gout_ref):
    d1 = d1_ref[...]
    agg = agg_ref[0] + agg_ref[1]
    pre = (1.0 - ALPHA) * d1 * (agg + g_ref[...]) + ALPHA * h0_ref[...]
    h = jnp.dot(pre, m_ref[...], preferred_element_type=jnp.float32)
    gout_ref[...] = d1 * h


def _dense_update(agg2p, gp, h0p, d1p, m4):
    grid = (NP4 + _BLKP - 1) // _BLKP
    return pl.pallas_call(
        _dense_body,
        grid=(grid,),
        in_specs=[
            pl.BlockSpec((2, _BLKP, 128), lambda i: (0, i, 0)),
            pl.BlockSpec((_BLKP, 128), lambda i: (i, 0)),
            pl.BlockSpec((_BLKP, 128), lambda i: (i, 0)),
            pl.BlockSpec((_BLKP, 128), lambda i: (i, 0)),
            pl.BlockSpec((128, 128), lambda i: (0, 0)),
        ],
        out_specs=pl.BlockSpec((_BLKP, 128), lambda i: (i, 0)),
        out_shape=jax.ShapeDtypeStruct((NP4, 128), jnp.float32),
    )(agg2p, gp, h0p, d1p, m4)


def _final_body(g_ref, s_ref, w_ref, b_ref, out_ref):
    h = g_ref[...] * s_ref[...]
    out_ref[...] = jnp.dot(h, w_ref[...], preferred_element_type=jnp.float32) + b_ref[...]


def _final(gp, ssqp, w4, b_lin):
    grid = (NP4 + _BLKP - 1) // _BLKP
    return pl.pallas_call(
        _final_body,
        grid=(grid,),
        in_specs=[
            pl.BlockSpec((_BLKP, 128), lambda i: (i, 0)),
            pl.BlockSpec((_BLKP, 128), lambda i: (i, 0)),
            pl.BlockSpec((128, 4), lambda i: (0, 0)),
            pl.BlockSpec((1, 1), lambda i: (0, 0)),
        ],
        out_specs=pl.BlockSpec((_BLKP, 4), lambda i: (i, 0)),
        out_shape=jax.ShapeDtypeStruct((NP4, 4), jnp.float32),
    )(gp, ssqp, w4, b_lin.reshape(1, 1))


def kernel(x, edge_index, w_steam, b_steam, w_blocks, w_lin, b_lin):
    row = edge_index[0]
    col = edge_index[1]

    npade = EP - E
    rowp = jnp.concatenate([row, jnp.zeros((npade,), jnp.int32)])
    colp = jnp.concatenate(
        [col, NPAD + (jnp.arange(npade, dtype=jnp.int32) % 64)])
    rowi2d = rowp.reshape(EP // 128, 128)
    coli2d = colp.reshape(EP // 128, 128)
    z = jnp.zeros((NROWS_T, NCH), jnp.float32)

    spmm = _make_spmm()

    aggd = spmm(rowi2d, coli2d, jnp.ones((NPAD, NCH), jnp.float32), z)
    deg = aggd[0, :N, 0] + aggd[1, :N, 0] + 1.0
    dinv = jnp.pad(lax.rsqrt(deg), (0, NPAD - N))
    d1p = jnp.repeat(dinv, NCH).reshape(NP4, 128)
    ssqp = jnp.repeat(jnp.pad(jnp.sqrt(deg), (0, NPAD - N)), NCH).reshape(NP4, 128)

    betas = jnp.log(THETA / jnp.arange(1, NUM_BLOCKS + 1, dtype=jnp.float32) + 1.0)
    eye = jnp.eye(NCH, dtype=jnp.float32)
    ms = (1.0 - betas)[:, None, None] * eye[None] + betas[:, None, None] * w_blocks
    eye4 = jnp.eye(4, dtype=jnp.float32)
    m4s = jax.vmap(lambda m: jnp.kron(eye4, m))(ms)
    w4 = jnp.kron(eye4, w_lin)

    h0 = _steam(x, w_steam, b_steam)
    h0p = jnp.pad(h0, ((0, NPAD - N), (0, 0))).reshape(NP4, 128)
    gp = d1p * h0p

    for l in range(NUM_BLOCKS):
        aggf = spmm(rowi2d, coli2d, gp.reshape(NPAD, NCH), z)
        aggp = aggf.reshape(2, NP4, 128)
        gp = _dense_update(aggp, gp, h0p, d1p, m4s[l])

    out4 = _final(gp, ssqp, w4, b_lin)
    return out4.reshape(NPAD, 1)[:N]


# speedup vs baseline: 56.5617x; 1.0248x over previous
import functools

import jax
import jax.numpy as jnp
from jax import lax
from jax.experimental import pallas as pl
from jax.experimental.pallas import tpu as pltpu
from jax.experimental.pallas import tpu_sc as plsc

NUM_BLOCKS = 64
NCH = 32
ALPHA = 0.1
THETA = 0.5

N = 50000
E = 1600000
NT = 32
NPAD = 50048
NP4 = NPAD // 4
NROWS_T = NPAD // 16
NACC = NPAD + 64
CH = 256
CHB = CH // 128
SUP = 7
CPT0 = 224
CPT1 = 168
NCHUNKS = 16 * (CPT0 + CPT1)
EP = NCHUNKS * CH
SROWS = SUP * CHB

_BLKP = 1024


def _spmm_body(rowi_hbm, coli_hbm, g_hbm, z_hbm, out_hbm,
               rowv, colv, stage0, stage1, acc,
               gsem0, gsem1, ssem0, ssem1, isem0, isem1):
    c = lax.axis_index("c")
    s = lax.axis_index("s")

    cbase = jnp.where(c == 0, s * CPT0, 16 * CPT0 + s * CPT1)
    npairs = jnp.where(c == 0, CPT0 // (2 * SUP), CPT1 // (2 * SUP))
    stages = (stage0, stage1)
    gsems = (gsem0, gsem1)
    ssems = (ssem0, ssem1)
    isems = (isem0, isem1)
    max_sb = (NCHUNKS - SUP) * CHB

    def idx_refs(u, b):
        sb = jnp.minimum((cbase + u * SUP) * CHB, max_sb)
        sb = pl.multiple_of(sb, 2)
        return [(rowi_hbm.at[pl.ds(sb, SROWS)],
                 rowv.at[pl.ds(b * SROWS, SROWS)]),
                (coli_hbm.at[pl.ds(sb, SROWS)],
                 colv.at[pl.ds(b * SROWS, SROWS)])]

    def fire_idx(u, b):
        for src, dst in idx_refs(u, b):
            pltpu.async_copy(src, dst, isems[b])

    def wait_idx(u, b):
        for src, dst in idx_refs(u, b):
            pltpu.make_async_copy(src, dst, isems[b]).wait()

    def scatter_refs(k):
        return [(stages[k % 2].at[pl.ds(j * 128, 128)],
                 acc.at[colv.at[j]]) for j in range(CHB)]

    def wait_scatter(k):
        for src, dst in scatter_refs(k):
            pltpu.make_async_copy(src, dst, ssems[k % 2]).wait()

    fire_idx(0, 0)

    @pl.when(c == 0)
    def _():
        pltpu.sync_copy(g_hbm.at[pl.ds(s * NROWS_T, NROWS_T)],
                        acc.at[pl.ds(s * NROWS_T, NROWS_T)])

    @pl.when(c != 0)
    def _():
        pltpu.sync_copy(z_hbm, acc.at[pl.ds(s * NROWS_T, NROWS_T)])
    plsc.subcore_barrier()

    def pair_body(p, carry):
        u0 = p * 2

        def fire_gather(k):
            ib = k // SUP
            return [pltpu.async_copy(
                g_hbm.at[rowv.at[ib * SROWS + (k - ib * SUP) * CHB + j]],
                stages[k % 2].at[pl.ds(j * 128, 128)], gsems[k % 2])
                for j in range(CHB)]

        def fire_scatter(k):
            ib = k // SUP
            return [pltpu.async_copy(
                stages[k % 2].at[pl.ds(j * 128, 128)],
                acc.at[colv.at[ib * SROWS + (k - ib * SUP) * CHB + j]],
                ssems[k % 2], add=True)
                for j in range(CHB)]

        wait_idx(u0, 0)
        fire_idx(u0 + 1, 1)
        @pl.when(p > 0)
        def _():
            wait_scatter(0)
        g_pending = fire_gather(0)
        s_pending = [None, None]
        for k in range(2 * SUP):
            if k == SUP:
                wait_idx(u0 + 1, 1)
                fire_idx(u0 + 2, 0)
            if k < 2 * SUP - 1:
                nb = (k + 1) % 2
                if k == 0:
                    @pl.when(p > 0)
                    def _():
                        wait_scatter(1)
                elif s_pending[nb] is not None:
                    for cp in s_pending[nb]:
                        cp.wait()
                    s_pending[nb] = None
                g_next = fire_gather(k + 1)
            for cp in g_pending:
                cp.wait()
            s_pending[k % 2] = fire_scatter(k)
            if k < 2 * SUP - 1:
                g_pending = g_next
        return carry

    lax.fori_loop(0, npairs, pair_body, 0)
    wait_scatter(0)
    wait_scatter(1)
    wait_idx(0, 0)
    plsc.subcore_barrier()

    pltpu.sync_copy(acc.at[pl.ds(s * NROWS_T, NROWS_T)],
                    out_hbm.at[c, pl.ds(s * NROWS_T, NROWS_T)])


def _make_spmm():
    mesh = plsc.VectorSubcoreMesh(core_axis_name="c", subcore_axis_name="s")
    return pl.kernel(
        _spmm_body,
        out_type=jax.ShapeDtypeStruct((2, NPAD, NCH), jnp.float32),
        mesh=mesh,
        compiler_params=pltpu.CompilerParams(use_tc_tiling_on_sc=False),
        scratch_types=[
            pltpu.VMEM((2 * SROWS, 128), jnp.int32),
            pltpu.VMEM((2 * SROWS, 128), jnp.int32),
            pltpu.VMEM((CH, NCH), jnp.float32),
            pltpu.VMEM((CH, NCH), jnp.float32),
            pltpu.VMEM_SHARED((NACC, NCH), jnp.float32),
            pltpu.SemaphoreType.DMA,
            pltpu.SemaphoreType.DMA,
            pltpu.SemaphoreType.DMA,
            pltpu.SemaphoreType.DMA,
            pltpu.SemaphoreType.DMA,
            pltpu.SemaphoreType.DMA,
        ],
    )


def _steam_body(x_ref, w_ref, b_ref, h0_ref):
    h = jnp.dot(x_ref[...], w_ref[...], preferred_element_type=jnp.float32)
    h0_ref[...] = h + b_ref[...]


def _steam(x, w_steam, b_steam):
    n = x.shape[0]
    blk = 2048
    grid = (n + blk - 1) // blk
    return pl.pallas_call(
        _steam_body,
        grid=(grid,),
        in_specs=[
            pl.BlockSpec((blk, 128), lambda i: (i, 0)),
            pl.BlockSpec((128, NCH), lambda i: (0, 0)),
            pl.BlockSpec((1, NCH), lambda i: (0, 0)),
        ],
        out_specs=pl.BlockSpec((blk, NCH), lambda i: (i, 0)),
        out_shape=jax.ShapeDtypeStruct((n, NCH), jnp.float32),
    )(x, w_steam, b_steam.reshape(1, NCH))


def _dense_body(agg_ref, h0_ref, d1_ref, m_ref, gout_ref):
    d1 = d1_ref[...]
    agg = agg_ref[0] + agg_ref[1]
    pre = (1.0 - ALPHA) * d1 * agg + ALPHA * h0_ref[...]
    h = jnp.dot(pre, m_ref[...], preferred_element_type=jnp.float32)
    gout_ref[...] = d1 * h


def _dense_update(agg2p, h0p, d1p, m4):
    grid = (NP4 + _BLKP - 1) // _BLKP
    return pl.pallas_call(
        _dense_body,
        grid=(grid,),
        in_specs=[
            pl.BlockSpec((2, _BLKP, 128), lambda i: (0, i, 0)),
            pl.BlockSpec((_BLKP, 128), lambda i: (i, 0)),
            pl.BlockSpec((_BLKP, 128), lambda i: (i, 0)),
            pl.BlockSpec((128, 128), lambda i: (0, 0)),
        ],
        out_specs=pl.BlockSpec((_BLKP, 128), lambda i: (i, 0)),
        out_shape=jax.ShapeDtypeStruct((NP4, 128), jnp.float32),
    )(agg2p, h0p, d1p, m4)


def _final_body(g_ref, s_ref, w_ref, b_ref, out_ref):
    h = g_ref[...] * s_ref[...]
    out_ref[...] = jnp.dot(h, w_ref[...], preferred_element_type=jnp.float32) + b_ref[...]


def _final(gp, ssqp, w4, b_lin):
    grid = (NP4 + _BLKP - 1) // _BLKP
    return pl.pallas_call(
        _final_body,
        grid=(grid,),
        in_specs=[
            pl.BlockSpec((_BLKP, 128), lambda i: (i, 0)),
            pl.BlockSpec((_BLKP, 128), lambda i: (i, 0)),
            pl.BlockSpec((128, 4), lambda i: (0, 0)),
            pl.BlockSpec((1, 1), lambda i: (0, 0)),
        ],
        out_specs=pl.BlockSpec((_BLKP, 4), lambda i: (i, 0)),
        out_shape=jax.ShapeDtypeStruct((NP4, 4), jnp.float32),
    )(gp, ssqp, w4, b_lin.reshape(1, 1))


def kernel(x, edge_index, w_steam, b_steam, w_blocks, w_lin, b_lin):
    row = edge_index[0]
    col = edge_index[1]

    npade = EP - E
    rowp = jnp.concatenate([row, jnp.zeros((npade,), jnp.int32)])
    colp = jnp.concatenate(
        [col, NPAD + (jnp.arange(npade, dtype=jnp.int32) % 64)])
    rowi2d = rowp.reshape(EP // 128, 128)
    coli2d = colp.reshape(EP // 128, 128)
    z = jnp.zeros((NROWS_T, NCH), jnp.float32)

    spmm = _make_spmm()

    aggd = spmm(rowi2d, coli2d, jnp.ones((NPAD, NCH), jnp.float32), z)
    deg = aggd[0, :N, 0] + aggd[1, :N, 0]
    dinv = jnp.pad(lax.rsqrt(deg), (0, NPAD - N))
    d1p = jnp.repeat(dinv, NCH).reshape(NP4, 128)
    ssqp = jnp.repeat(jnp.pad(jnp.sqrt(deg), (0, NPAD - N)), NCH).reshape(NP4, 128)

    betas = jnp.log(THETA / jnp.arange(1, NUM_BLOCKS + 1, dtype=jnp.float32) + 1.0)
    eye = jnp.eye(NCH, dtype=jnp.float32)
    ms = (1.0 - betas)[:, None, None] * eye[None] + betas[:, None, None] * w_blocks
    eye4 = jnp.eye(4, dtype=jnp.float32)
    m4s = jax.vmap(lambda m: jnp.kron(eye4, m))(ms)
    w4 = jnp.kron(eye4, w_lin)

    h0 = _steam(x, w_steam, b_steam)
    h0p = jnp.pad(h0, ((0, NPAD - N), (0, 0))).reshape(NP4, 128)
    gp = d1p * h0p

    for l in range(NUM_BLOCKS):
        aggf = spmm(rowi2d, coli2d, gp.reshape(NPAD, NCH), z)
        aggp = aggf.reshape(2, NP4, 128)
        gp = _dense_update(aggp, h0p, d1p, m4s[l])

    out4 = _final(gp, ssqp, w4, b_lin)
    return out4.reshape(NPAD, 1)[:N]
